# baseline (speedup 1.0000x reference)
import functools

import jax
import jax.numpy as jnp
from jax import lax
from jax.experimental import pallas as pl
from jax.experimental.pallas import tpu as pltpu
from jax.experimental.pallas import tpu_sc as plsc

DIM = 64
SCALE = 8.0
NC, NS = 2, 16
NW = NC * NS
B = 4096 * 200
BPW = B // NW
IW = 128
K = 8
C = K * IW
NCH = B // C // NW


def _emb_body(x_hbm, tab_hbm, out_hbm, idx_v, rows_v, sem):
    wid = lax.axis_index("s") * NC + lax.axis_index("c")

    def chunk(i, carry):
        g = i * NW + wid
        off = g * C
        pltpu.sync_copy(x_hbm.at[pl.ds(g * K, K)], idx_v)
        copies = [
            pltpu.make_async_copy(
                tab_hbm.at[idx_v.at[j]],
                rows_v.at[pl.ds(j * IW, IW)],
                sem,
            )
            for j in range(K)
        ]
        for cp in copies:
            cp.start()
        for cp in copies:
            cp.wait()

        def scale_row(r, acc):
            for c4 in range(DIM // 16):
                sl = pl.ds(c4 * 16, 16)
                rows_v[r, sl] = rows_v[r, sl] * SCALE
            return acc

        lax.fori_loop(0, C, scale_row, 0)
        pltpu.sync_copy(rows_v, out_hbm.at[pl.ds(off, C)])
        return carry

    lax.fori_loop(0, NCH, chunk, 0)


@jax.jit
def kernel(x, table):
    xf = x.reshape(B // IW, IW).astype(jnp.int32)
    mesh = plsc.VectorSubcoreMesh(core_axis_name="c", subcore_axis_name="s")
    out = pl.kernel(
        _emb_body,
        out_type=jax.ShapeDtypeStruct((B, DIM), jnp.float32),
        mesh=mesh,
        compiler_params=pltpu.CompilerParams(use_tc_tiling_on_sc=False),
        scratch_types=[
            pltpu.VMEM((K, IW), jnp.int32),
            pltpu.VMEM((C, DIM), jnp.float32),
            pltpu.SemaphoreType.DMA,
        ],
    )(xf, table)
    return out.reshape(x.shape[0], x.shape[1], DIM)


# speedup vs baseline: 1.0075x; 1.0075x over previous
import functools

import jax
import jax.numpy as jnp
from jax import lax
from jax.experimental import pallas as pl
from jax.experimental.pallas import tpu as pltpu
from jax.experimental.pallas import tpu_sc as plsc

DIM = 64
SCALE = 8.0
NC, NS = 2, 16
NW = NC * NS
NBATCH = 4096
SEQ = 200
SPLITS = ((0, 104), (104, 96))
WB = NBATCH // NW
NB = 4
NSC = WB // NB


def _emb_body(x_hbm, tab_hbm, out_hbm, idx_all, rows_v, sem):
    wid = lax.axis_index("s") * NC + lax.axis_index("c")
    b0 = wid * WB
    pltpu.sync_copy(x_hbm.at[pl.ds(b0, WB)], idx_all)

    def sub_chunk(s, carry):
        copies = []
        for nb in range(NB):
            bi = s * NB + nb
            for off, ln in SPLITS:
                copies.append(
                    pltpu.make_async_copy(
                        tab_hbm.at[idx_all.at[bi, pl.ds(off, ln)]],
                        rows_v.at[nb, pl.ds(off, ln)],
                        sem,
                    )
                )
        for cp in copies:
            cp.start()
        for cp in copies:
            cp.wait()

        for nb in range(NB):
            def scale_row(r, acc, nb=nb):
                for c4 in range(DIM // 16):
                    sl = pl.ds(c4 * 16, 16)
                    rows_v[nb, r, sl] = rows_v[nb, r, sl] * SCALE
                return acc

            lax.fori_loop(0, SEQ, scale_row, 0)

        pltpu.sync_copy(rows_v, out_hbm.at[pl.ds(b0 + s * NB, NB)])
        return carry

    lax.fori_loop(0, NSC, sub_chunk, 0)


@jax.jit
def kernel(x, table):
    mesh = plsc.VectorSubcoreMesh(core_axis_name="c", subcore_axis_name="s")
    return pl.kernel(
        _emb_body,
        out_type=jax.ShapeDtypeStruct((NBATCH, SEQ, DIM), jnp.float32),
        mesh=mesh,
        compiler_params=pltpu.CompilerParams(use_tc_tiling_on_sc=False),
        scratch_types=[
            pltpu.VMEM((WB, SEQ), jnp.int32),
            pltpu.VMEM((NB, SEQ, DIM), jnp.float32),
            pltpu.SemaphoreType.DMA,
        ],
    )(x.astype(jnp.int32), table)
